# Initial kernel scaffold; baseline (speedup 1.0000x reference)
import jax
import jax.numpy as jnp
from jax.experimental import pallas as pl


def kernel(in_idx, tok_table, pos_table):
    raise NotImplementedError("write your pallas kernel here")


# speedup vs baseline: 1.0317x; 1.0317x over previous
import functools

import jax
import jax.numpy as jnp
from jax import lax
from jax.experimental import pallas as pl
from jax.experimental.pallas import tpu as pltpu
from jax.experimental.pallas import tpu_sc as plsc

_B, _S, _EMB = 4, 2048, 768
_N = _B * _S
_NC, _NS = 2, 16
_NW = _NC * _NS
_RPW = _N // _NW
_CH = 64
_NCH = _RPW // _CH
_LANES = 16
_VECS = _EMB // _LANES

_mesh = plsc.VectorSubcoreMesh(core_axis_name="c", subcore_axis_name="s")


@functools.partial(
    pl.kernel,
    mesh=_mesh,
    out_type=jax.ShapeDtypeStruct((_N, _EMB), jnp.float32),
    scratch_types=[
        pltpu.VMEM((_NCH, _CH), jnp.int32),
        pltpu.VMEM((_CH, _EMB), jnp.float32),
        pltpu.VMEM((_CH, _EMB), jnp.float32),
        pltpu.SemaphoreType.DMA,
        pltpu.SemaphoreType.DMA,
    ],
)
def _embed(idx_hbm, tok_hbm, pos_hbm, out_hbm, idx_v, buf, pbuf, sem_g, sem_p):
    wid = lax.axis_index("s") * _NC + lax.axis_index("c")
    base = wid * _RPW
    pltpu.sync_copy(idx_hbm.at[wid], idx_v)
    for ch in range(_NCH):
        row0 = base + ch * _CH
        s0 = lax.rem(row0, _S)
        g = pltpu.async_copy(tok_hbm.at[idx_v.at[ch]], buf, sem_g)
        p = pltpu.async_copy(pos_hbm.at[pl.ds(s0, _CH)], pbuf, sem_p)
        g.wait()
        p.wait()

        def add_row(r, carry):
            for c in range(_VECS):
                sl = pl.ds(c * _LANES, _LANES)
                buf[r, sl] = buf[r, sl] + pbuf[r, sl]
            return carry

        lax.fori_loop(0, _CH, add_row, 0)
        pltpu.sync_copy(buf, out_hbm.at[pl.ds(row0, _CH)])


def kernel(in_idx, tok_table, pos_table):
    idx = in_idx.reshape(_NW, _NCH, _CH).astype(jnp.int32)
    out = _embed(idx, tok_table, pos_table)
    return out.reshape(_B, _S, _EMB)
